# baseline (speedup 1.0000x reference)
import functools

import jax
import jax.numpy as jnp
from jax import lax
from jax.experimental import pallas as pl
from jax.experimental.pallas import tpu as pltpu
from jax.experimental.pallas import tpu_sc as plsc

N = 10000
E = 320000
DIN = 128
HID = 16
HEADS = 8
DOUT = 16
NG = 64
NEG_SLOPE = 0.2

NPAD = 10112
RPT = NPAD // 16
TOTE = E + N
NW = 32

C1, SUB1, CPW1 = 80, 1, 129
C2, SUB2, CPW2 = 384, 3, 27
TOTP1 = NW * CPW1 * C1
TOTP2 = NW * CPW2 * C2


def _stage_a_body(x_ref, w_ref, ms_ref, md_ref, h_ref, ts_ref, td_ref):
    h = jnp.dot(x_ref[...], w_ref[...], preferred_element_type=jnp.float32)
    h_ref[...] = h
    ts_ref[...] = jnp.dot(h, ms_ref[...], preferred_element_type=jnp.float32)
    td_ref[...] = jnp.dot(h, md_ref[...], preferred_element_type=jnp.float32)


def _stage_a(x, W, Ms, Md):
    n = x.shape[0]
    dout = W.shape[1]
    return pl.pallas_call(
        _stage_a_body,
        out_shape=(
            jax.ShapeDtypeStruct((n, dout), jnp.float32),
            jax.ShapeDtypeStruct((n, 16), jnp.float32),
            jax.ShapeDtypeStruct((n, 16), jnp.float32),
        ),
    )(x, W, Ms, Md)


def _make_sc_edge(D, H, C, SUB, CPW):
    CS = C // SUB
    MERGE = (D + 16) <= 128
    DW = D + 16 if MERGE else D
    mesh = plsc.VectorSubcoreMesh(core_axis_name="c", subcore_axis_name="s")

    def body(sd_h, tabS_h, tabD_h, htab_h, z16_h, zD_h, *refs):
        if MERGE:
            (acc_o, sd, dsc, bufS, bufD, hbuf, wbuf, acc_sh, *sems) = refs
        else:
            (den_o, acc_o, sd, dsc, bufS, bufD, hbuf, den_sh, acc_sh,
             *sems) = refs
        cid = lax.axis_index("c")
        sid = lax.axis_index("s")
        wid = cid * 16 + sid
        row0 = pl.multiple_of(sid * RPT, 8)
        pltpu.sync_copy(zD_h.at[pl.ds(row0, RPT)], acc_sh.at[pl.ds(row0, RPT)])
        if not MERGE:
            pltpu.sync_copy(z16_h.at[pl.ds(row0, RPT)],
                            den_sh.at[pl.ds(row0, RPT)])
        plsc.subcore_barrier()

        lane = lax.iota(jnp.int32, 16)
        lane_ok = lane < H
        ck0 = wid * CPW
        isems = sems[0:3]
        dsems = sems[3:6]
        gsems = sems[6:9]
        ssems = sems[9:12]

        def idx_start(k, slot):
            pltpu.async_copy(sd_h.at[ck0 + k], sd.at[slot], isems[slot])

        def idx_wait(slot):
            pltpu.make_async_copy(sd_h.at[0], sd.at[slot], isems[slot]).wait()

        def dsc_start(k, slot):
            pltpu.async_copy(sd_h.at[ck0 + k, 1], dsc.at[slot], dsems[slot])

        def dsc_wait(slot):
            pltpu.make_async_copy(sd_h.at[0, 1], dsc.at[slot],
                                  dsems[slot]).wait()

        def gathers_start(slot):
            for s in range(SUB):
                r = pl.ds(s * CS, CS)
                pltpu.async_copy(tabS_h.at[sd.at[slot, 0, s]],
                                 bufS.at[slot, r], gsems[slot])
                pltpu.async_copy(tabD_h.at[sd.at[slot, 1, s]],
                                 bufD.at[slot, r], gsems[slot])
                pltpu.async_copy(htab_h.at[sd.at[slot, 0, s]],
                                 hbuf.at[slot, r], gsems[slot])

        def gathers_wait(slot):
            for s in range(SUB):
                r = pl.ds(s * CS, CS)
                pltpu.make_async_copy(tabS_h.at[sd.at[slot, 0, s]],
                                      bufS.at[slot, r], gsems[slot]).wait()
                pltpu.make_async_copy(tabD_h.at[sd.at[slot, 1, s]],
                                      bufD.at[slot, r], gsems[slot]).wait()
                pltpu.make_async_copy(htab_h.at[sd.at[slot, 0, s]],
                                      hbuf.at[slot, r], gsems[slot]).wait()

        if MERGE:
            def scatters_start(slot):
                for s in range(SUB):
                    pltpu.async_copy(wbuf.at[slot, pl.ds(s * CS, CS)],
                                     acc_sh.at[dsc.at[slot, s]],
                                     ssems[slot], add=True)

            def scatters_wait(slot):
                for s in range(SUB):
                    pltpu.make_async_copy(wbuf.at[slot, pl.ds(s * CS, CS)],
                                          acc_sh.at[dsc.at[slot, s]],
                                          ssems[slot]).wait()

            def compute(slot):
                def cl(i, c2):
                    e = bufS[slot, i, :] + bufD[slot, i, :]
                    e = jnp.where(e > 0.0, e, NEG_SLOPE * e)
                    ex = jnp.where(lane_ok, jnp.exp(e), 0.0)
                    wbuf[slot, i, pl.ds(D, 16)] = ex
                    for j in range(D // 16):
                        hv = hbuf[slot, i, pl.ds(j * 16, 16)]
                        wbuf[slot, i, pl.ds(j * 16, 16)] = hv * jnp.full(
                            (16,), ex[j], jnp.float32)
                    return c2

                lax.fori_loop(0, C, cl, 0, unroll=4)
        else:
            def scatters_start(slot):
                for s in range(SUB):
                    r = pl.ds(s * CS, CS)
                    pltpu.async_copy(bufS.at[slot, r],
                                     den_sh.at[dsc.at[slot, s]],
                                     ssems[slot], add=True)
                    pltpu.async_copy(hbuf.at[slot, r],
                                     acc_sh.at[dsc.at[slot, s]],
                                     ssems[slot], add=True)

            def scatters_wait(slot):
                for s in range(SUB):
                    r = pl.ds(s * CS, CS)
                    pltpu.make_async_copy(bufS.at[slot, r],
                                          den_sh.at[dsc.at[slot, s]],
                                          ssems[slot]).wait()
                    pltpu.make_async_copy(hbuf.at[slot, r],
                                          acc_sh.at[dsc.at[slot, s]],
                                          ssems[slot]).wait()

            def compute(slot):
                def cl(i, c2):
                    e = bufS[slot, i, :] + bufD[slot, i, :]
                    e = jnp.where(e > 0.0, e, NEG_SLOPE * e)
                    ex = jnp.where(lane_ok, jnp.exp(e), 0.0)
                    bufS[slot, i, :] = ex
                    for j in range(D // 16):
                        hv = hbuf[slot, i, pl.ds(j * 16, 16)]
                        hbuf[slot, i, pl.ds(j * 16, 16)] = hv * jnp.full(
                            (16,), ex[j], jnp.float32)
                    return c2

                lax.fori_loop(0, C, cl, 0, unroll=4)

        def handle(k, slot):
            sn = (slot + 1) % 3
            sp = (slot + 2) % 3

            @pl.when(k + 1 < CPW)
            def _():
                idx_wait(sn)
                gathers_start(sn)

            gathers_wait(slot)

            @pl.when(k + 2 < CPW)
            def _():
                idx_start(k + 2, sp)

            compute(slot)

            @pl.when(k >= 1)
            def _():
                scatters_wait(sp)

            @pl.when(k + 2 < CPW)
            def _():
                dsc_start(k + 2, sp)

            dsc_wait(slot)
            scatters_start(slot)

        idx_start(0, 0)
        idx_start(1, 1)
        dsc_start(0, 0)
        dsc_start(1, 1)
        idx_wait(0)
        gathers_start(0)

        def triple(g, carry):
            handle(3 * g, 0)
            handle(3 * g + 1, 1)
            handle(3 * g + 2, 2)
            return carry

        lax.fori_loop(0, CPW // 3, triple, 0)
        scatters_wait((CPW - 1) % 3)

        plsc.subcore_barrier()
        pltpu.sync_copy(acc_sh.at[pl.ds(row0, RPT)],
                        acc_o.at[cid, pl.ds(row0, RPT)])
        if not MERGE:
            pltpu.sync_copy(den_sh.at[pl.ds(row0, RPT)],
                            den_o.at[cid, pl.ds(row0, RPT)])

    out_type = [jax.ShapeDtypeStruct((2, NPAD, DW), jnp.float32)]
    scratch = [
        pltpu.VMEM((3, 2, SUB, CS), jnp.int32),
        pltpu.VMEM((3, SUB, CS), jnp.int32),
        pltpu.VMEM((3, C, 16), jnp.float32),
        pltpu.VMEM((3, C, 16), jnp.float32),
        pltpu.VMEM((3, C, D), jnp.float32),
    ]
    if MERGE:
        scratch += [
            pltpu.VMEM((3, C, DW), jnp.float32),
            pltpu.VMEM_SHARED((NPAD, DW), jnp.float32),
        ]
    else:
        out_type = [jax.ShapeDtypeStruct((2, NPAD, 16), jnp.float32)] + out_type
        scratch += [
            pltpu.VMEM_SHARED((NPAD, 16), jnp.float32),
            pltpu.VMEM_SHARED((NPAD, D), jnp.float32),
        ]
    scratch += [pltpu.SemaphoreType.DMA] * 12

    return pl.kernel(
        body,
        out_type=out_type,
        mesh=mesh,
        scratch_types=scratch,
        compiler_params=pltpu.CompilerParams(use_tc_tiling_on_sc=False),
    )


def _stage_b_body(den_r, acc_r, R_r, b1_r, W2_r, Ms_r, Md_r, h2_o, ts_o, td_o):
    densum = den_r[0] + den_r[1]
    accsum = acc_r[0] + acc_r[1]
    dexp = jnp.dot(densum, R_r[...], preferred_element_type=jnp.float32)
    g1 = accsum / (dexp + 1e-16) + b1_r[...]
    g1 = jnp.where(g1 > 0.0, g1, jnp.exp(jnp.minimum(g1, 0.0)) - 1.0)
    h2 = jnp.dot(g1, W2_r[...], preferred_element_type=jnp.float32)
    h2_o[...] = h2
    ts_o[...] = jnp.dot(h2, Ms_r[...], preferred_element_type=jnp.float32)
    td_o[...] = jnp.dot(h2, Md_r[...], preferred_element_type=jnp.float32)


def _stage_b(den1, acc1, R, b1r, W2, Ms2, Md2):
    return pl.pallas_call(
        _stage_b_body,
        out_shape=(
            jax.ShapeDtypeStruct((NPAD, 16), jnp.float32),
            jax.ShapeDtypeStruct((NPAD, 16), jnp.float32),
            jax.ShapeDtypeStruct((NPAD, 16), jnp.float32),
        ),
    )(den1, acc1, R, b1r, W2, Ms2, Md2)


def _stage_c_body(acc_r, b2_r, batch_r, B0_r, out_o):
    accsum = acc_r[0] + acc_r[1]
    densum = accsum[:, 16:32]
    dl = jnp.dot(densum, B0_r[...], preferred_element_type=jnp.float32)
    g2full = accsum[:, 0:16] / (dl + 1e-16)
    g2 = g2full[0:N, :] + b2_r[...]
    gid = lax.broadcasted_iota(jnp.int32, (NG, N), 0)
    bm = jnp.where(gid == batch_r[...], 1.0, 0.0)
    pooled = jnp.dot(bm, g2, preferred_element_type=jnp.float32)
    cnt = jnp.sum(bm, axis=1, keepdims=True)
    pooled = pooled / jnp.maximum(cnt, 1.0)
    m = jnp.max(pooled, axis=1, keepdims=True)
    p = pooled - m
    lse = jnp.log(jnp.sum(jnp.exp(p), axis=1, keepdims=True))
    out_o[...] = p - lse


def _stage_c(acc2, b2r, batch2, B0):
    return pl.pallas_call(
        _stage_c_body,
        out_shape=jax.ShapeDtypeStruct((NG, DOUT), jnp.float32),
    )(acc2, b2r, batch2, B0)


def _blockdiag(a):
    H_, ch = a.shape
    eye = jnp.eye(H_, dtype=jnp.float32)
    return (a[:, :, None] * eye[:, None, :]).reshape(H_ * ch, H_)


def _sel_pair(a_src, a_dst):
    bs, bd = _blockdiag(a_src), _blockdiag(a_dst)
    H_ = a_src.shape[0]
    pad = jnp.zeros((bs.shape[0], 16 - 2 * H_), jnp.float32)
    Ms = jnp.concatenate([bs, bd, pad], axis=1)
    Md = jnp.concatenate([bd, bs, pad], axis=1)
    return Ms, Md


def _pad_rows(a, rows):
    return jnp.pad(a, ((0, rows - a.shape[0]),) + ((0, 0),) * (a.ndim - 1))


def _chunked_edges(edge_index, totp, c, sub):
    loop = jnp.arange(N, dtype=jnp.int32)
    padi = jnp.full((totp - TOTE,), N, jnp.int32)
    s = jnp.concatenate([edge_index[0].astype(jnp.int32), loop, padi])
    d = jnp.concatenate([edge_index[1].astype(jnp.int32), loop, padi])
    sd = jnp.stack([s, d], 0).reshape(2, totp // c, sub, c // sub)
    return sd.transpose(1, 0, 2, 3)


def kernel(x, edge_index, batch, W1, a_src1, a_dst1, b1, W2, a_src2, a_dst2, b2):
    sd1 = _chunked_edges(edge_index, TOTP1, C1, SUB1)
    sd2 = _chunked_edges(edge_index, TOTP2, C2, SUB2)
    z16 = jnp.zeros((NPAD, 16), jnp.float32)
    z128 = jnp.zeros((NPAD, 128), jnp.float32)
    z32 = jnp.zeros((NPAD, 32), jnp.float32)

    Ms1, Md1 = _sel_pair(a_src1, a_dst1)
    h1, tS1, tD1 = _stage_a(x, W1, Ms1, Md1)
    den1, acc1 = _make_sc_edge(128, HEADS, C1, SUB1, CPW1)(
        sd1, _pad_rows(tS1, NPAD), _pad_rows(tD1, NPAD),
        _pad_rows(h1, NPAD), z16, z128)

    hsel = lax.broadcasted_iota(jnp.int32, (16, 128), 0)
    csel = lax.broadcasted_iota(jnp.int32, (16, 128), 1) // HID
    R = jnp.where(hsel == csel, 1.0, 0.0)
    Ms2, Md2 = _sel_pair(a_src2, a_dst2)
    h2, tS2, tD2 = _stage_b(den1, acc1, R, b1.reshape(1, 128), W2, Ms2, Md2)

    (acc2,) = _make_sc_edge(16, 1, C2, SUB2, CPW2)(sd2, tS2, tD2, h2, z16, z32)

    B0 = jnp.where(lax.broadcasted_iota(jnp.int32, (16, 16), 0) == 0, 1.0, 0.0)
    return _stage_c(acc2, b2.reshape(1, DOUT), batch.reshape(1, N), B0)


# speedup vs baseline: 1.0423x; 1.0423x over previous
import functools

import jax
import jax.numpy as jnp
from jax import lax
from jax.experimental import pallas as pl
from jax.experimental.pallas import tpu as pltpu
from jax.experimental.pallas import tpu_sc as plsc

N = 10000
E = 320000
DIN = 128
HID = 16
HEADS = 8
DOUT = 16
NG = 64
NEG_SLOPE = 0.2

NPAD = 10112
RPT = NPAD // 16
TOTE = E + N
NW = 32

C1, SUB1, CPW1 = 80, 1, 129
C2, SUB2, CPW2 = 384, 3, 27
TOTP1 = NW * CPW1 * C1
TOTP2 = NW * CPW2 * C2


def _stage_a_body(x_ref, w_ref, ms_ref, md_ref, h_ref, ts_ref, td_ref):
    h = jnp.dot(x_ref[...], w_ref[...], preferred_element_type=jnp.float32)
    h_ref[...] = h
    ts_ref[...] = jnp.dot(h, ms_ref[...], preferred_element_type=jnp.float32)
    td_ref[...] = jnp.dot(h, md_ref[...], preferred_element_type=jnp.float32)


def _stage_a(x, W, Ms, Md):
    n = x.shape[0]
    dout = W.shape[1]
    return pl.pallas_call(
        _stage_a_body,
        out_shape=(
            jax.ShapeDtypeStruct((n, dout), jnp.float32),
            jax.ShapeDtypeStruct((n, 16), jnp.float32),
            jax.ShapeDtypeStruct((n, 16), jnp.float32),
        ),
    )(x, W, Ms, Md)


def _make_sc_edge(D, H, C, SUB, CPW):
    CS = C // SUB
    MERGE = (D + 16) <= 128
    DW = D + 16 if MERGE else D
    mesh = plsc.VectorSubcoreMesh(core_axis_name="c", subcore_axis_name="s")

    def body(sd_h, tabS_h, tabD_h, htab_h, z16_h, zD_h, *refs):
        if MERGE:
            (acc_o, sd, dsc, bufS, bufD, hbuf, wbuf, acc_sh, *sems) = refs
        else:
            (den_o, acc_o, sd, dsc, bufS, bufD, hbuf, den_sh, acc_sh,
             *sems) = refs
        cid = lax.axis_index("c")
        sid = lax.axis_index("s")
        wid = cid * 16 + sid
        row0 = pl.multiple_of(sid * RPT, 8)
        pltpu.sync_copy(zD_h.at[pl.ds(row0, RPT)], acc_sh.at[pl.ds(row0, RPT)])
        if not MERGE:
            pltpu.sync_copy(z16_h.at[pl.ds(row0, RPT)],
                            den_sh.at[pl.ds(row0, RPT)])
        plsc.subcore_barrier()

        lane = lax.iota(jnp.int32, 16)
        lane_ok = lane < H
        ck0 = wid * CPW
        isems = sems[0:3]
        dsems = sems[3:6]
        gsems = sems[6:9]
        ssems = sems[9:12]

        def idx_start(k, slot):
            pltpu.async_copy(sd_h.at[ck0 + k], sd.at[slot], isems[slot])

        def idx_wait(slot):
            pltpu.make_async_copy(sd_h.at[0], sd.at[slot], isems[slot]).wait()

        def dsc_start(k, slot):
            pltpu.async_copy(sd_h.at[ck0 + k, 1], dsc.at[slot], dsems[slot])

        def dsc_wait(slot):
            pltpu.make_async_copy(sd_h.at[0, 1], dsc.at[slot],
                                  dsems[slot]).wait()

        def gathers_start(slot):
            for s in range(SUB):
                r = pl.ds(s * CS, CS)
                pltpu.async_copy(tabS_h.at[sd.at[slot, 0, s]],
                                 bufS.at[slot, r], gsems[slot])
                pltpu.async_copy(tabD_h.at[sd.at[slot, 1, s]],
                                 bufD.at[slot, r], gsems[slot])
                pltpu.async_copy(htab_h.at[sd.at[slot, 0, s]],
                                 hbuf.at[slot, r], gsems[slot])

        def gathers_wait(slot):
            for s in range(SUB):
                r = pl.ds(s * CS, CS)
                pltpu.make_async_copy(tabS_h.at[sd.at[slot, 0, s]],
                                      bufS.at[slot, r], gsems[slot]).wait()
                pltpu.make_async_copy(tabD_h.at[sd.at[slot, 1, s]],
                                      bufD.at[slot, r], gsems[slot]).wait()
                pltpu.make_async_copy(htab_h.at[sd.at[slot, 0, s]],
                                      hbuf.at[slot, r], gsems[slot]).wait()

        if MERGE:
            def scatters_start(slot):
                for s in range(SUB):
                    pltpu.async_copy(wbuf.at[slot, pl.ds(s * CS, CS)],
                                     acc_sh.at[dsc.at[slot, s]],
                                     ssems[slot], add=True)

            def scatters_wait(slot):
                for s in range(SUB):
                    pltpu.make_async_copy(wbuf.at[slot, pl.ds(s * CS, CS)],
                                          acc_sh.at[dsc.at[slot, s]],
                                          ssems[slot]).wait()

            def compute(slot):
                def cl(i, c2):
                    e = bufS[slot, i, :] + bufD[slot, i, :]
                    e = jnp.where(e > 0.0, e, NEG_SLOPE * e)
                    ex = jnp.where(lane_ok, jnp.exp(e), 0.0)
                    wbuf[slot, i, pl.ds(D, 16)] = ex
                    for j in range(D // 16):
                        hv = hbuf[slot, i, pl.ds(j * 16, 16)]
                        wbuf[slot, i, pl.ds(j * 16, 16)] = hv * jnp.full(
                            (16,), ex[j], jnp.float32)
                    return c2

                lax.fori_loop(0, C, cl, 0, unroll=2)
        else:
            def scatters_start(slot):
                for s in range(SUB):
                    r = pl.ds(s * CS, CS)
                    pltpu.async_copy(bufS.at[slot, r],
                                     den_sh.at[dsc.at[slot, s]],
                                     ssems[slot], add=True)
                    pltpu.async_copy(hbuf.at[slot, r],
                                     acc_sh.at[dsc.at[slot, s]],
                                     ssems[slot], add=True)

            def scatters_wait(slot):
                for s in range(SUB):
                    r = pl.ds(s * CS, CS)
                    pltpu.make_async_copy(bufS.at[slot, r],
                                          den_sh.at[dsc.at[slot, s]],
                                          ssems[slot]).wait()
                    pltpu.make_async_copy(hbuf.at[slot, r],
                                          acc_sh.at[dsc.at[slot, s]],
                                          ssems[slot]).wait()

            def compute(slot):
                def cl(i, c2):
                    e = bufS[slot, i, :] + bufD[slot, i, :]
                    e = jnp.where(e > 0.0, e, NEG_SLOPE * e)
                    ex = jnp.where(lane_ok, jnp.exp(e), 0.0)
                    bufS[slot, i, :] = ex
                    for j in range(D // 16):
                        hv = hbuf[slot, i, pl.ds(j * 16, 16)]
                        hbuf[slot, i, pl.ds(j * 16, 16)] = hv * jnp.full(
                            (16,), ex[j], jnp.float32)
                    return c2

                lax.fori_loop(0, C, cl, 0, unroll=2)

        def handle(k, slot):
            sn = (slot + 1) % 3
            sp = (slot + 2) % 3

            @pl.when(k + 1 < CPW)
            def _():
                idx_wait(sn)
                gathers_start(sn)

            gathers_wait(slot)

            @pl.when(k + 2 < CPW)
            def _():
                idx_start(k + 2, sp)

            compute(slot)

            @pl.when(k >= 1)
            def _():
                scatters_wait(sp)

            @pl.when(k + 2 < CPW)
            def _():
                dsc_start(k + 2, sp)

            dsc_wait(slot)
            scatters_start(slot)

        idx_start(0, 0)
        idx_start(1, 1)
        dsc_start(0, 0)
        dsc_start(1, 1)
        idx_wait(0)
        gathers_start(0)

        def triple(g, carry):
            handle(3 * g, 0)
            handle(3 * g + 1, 1)
            handle(3 * g + 2, 2)
            return carry

        lax.fori_loop(0, CPW // 3, triple, 0)
        scatters_wait((CPW - 1) % 3)

        plsc.subcore_barrier()
        pltpu.sync_copy(acc_sh.at[pl.ds(row0, RPT)],
                        acc_o.at[cid, pl.ds(row0, RPT)])
        if not MERGE:
            pltpu.sync_copy(den_sh.at[pl.ds(row0, RPT)],
                            den_o.at[cid, pl.ds(row0, RPT)])

    out_type = [jax.ShapeDtypeStruct((2, NPAD, DW), jnp.float32)]
    scratch = [
        pltpu.VMEM((3, 2, SUB, CS), jnp.int32),
        pltpu.VMEM((3, SUB, CS), jnp.int32),
        pltpu.VMEM((3, C, 16), jnp.float32),
        pltpu.VMEM((3, C, 16), jnp.float32),
        pltpu.VMEM((3, C, D), jnp.float32),
    ]
    if MERGE:
        scratch += [
            pltpu.VMEM((3, C, DW), jnp.float32),
            pltpu.VMEM_SHARED((NPAD, DW), jnp.float32),
        ]
    else:
        out_type = [jax.ShapeDtypeStruct((2, NPAD, 16), jnp.float32)] + out_type
        scratch += [
            pltpu.VMEM_SHARED((NPAD, 16), jnp.float32),
            pltpu.VMEM_SHARED((NPAD, D), jnp.float32),
        ]
    scratch += [pltpu.SemaphoreType.DMA] * 12

    return pl.kernel(
        body,
        out_type=out_type,
        mesh=mesh,
        scratch_types=scratch,
        compiler_params=pltpu.CompilerParams(use_tc_tiling_on_sc=False,
                                             skip_device_barrier=True),
    )


def _stage_b_body(den_r, acc_r, R_r, b1_r, W2_r, Ms_r, Md_r, h2_o, ts_o, td_o):
    densum = den_r[0] + den_r[1]
    accsum = acc_r[0] + acc_r[1]
    dexp = jnp.dot(densum, R_r[...], preferred_element_type=jnp.float32)
    g1 = accsum / (dexp + 1e-16) + b1_r[...]
    g1 = jnp.where(g1 > 0.0, g1, jnp.exp(jnp.minimum(g1, 0.0)) - 1.0)
    h2 = jnp.dot(g1, W2_r[...], preferred_element_type=jnp.float32)
    h2_o[...] = h2
    ts_o[...] = jnp.dot(h2, Ms_r[...], preferred_element_type=jnp.float32)
    td_o[...] = jnp.dot(h2, Md_r[...], preferred_element_type=jnp.float32)


def _stage_b(den1, acc1, R, b1r, W2, Ms2, Md2):
    return pl.pallas_call(
        _stage_b_body,
        out_shape=(
            jax.ShapeDtypeStruct((NPAD, 16), jnp.float32),
            jax.ShapeDtypeStruct((NPAD, 16), jnp.float32),
            jax.ShapeDtypeStruct((NPAD, 16), jnp.float32),
        ),
    )(den1, acc1, R, b1r, W2, Ms2, Md2)


def _stage_c_body(acc_r, b2_r, batch_r, B0_r, out_o):
    accsum = acc_r[0] + acc_r[1]
    densum = accsum[:, 16:32]
    dl = jnp.dot(densum, B0_r[...], preferred_element_type=jnp.float32)
    g2full = accsum[:, 0:16] / (dl + 1e-16)
    g2 = g2full[0:N, :] + b2_r[...]
    gid = lax.broadcasted_iota(jnp.int32, (NG, N), 0)
    bm = jnp.where(gid == batch_r[...], 1.0, 0.0)
    pooled = jnp.dot(bm, g2, preferred_element_type=jnp.float32)
    cnt = jnp.sum(bm, axis=1, keepdims=True)
    pooled = pooled / jnp.maximum(cnt, 1.0)
    m = jnp.max(pooled, axis=1, keepdims=True)
    p = pooled - m
    lse = jnp.log(jnp.sum(jnp.exp(p), axis=1, keepdims=True))
    out_o[...] = p - lse


def _stage_c(acc2, b2r, batch2, B0):
    return pl.pallas_call(
        _stage_c_body,
        out_shape=jax.ShapeDtypeStruct((NG, DOUT), jnp.float32),
    )(acc2, b2r, batch2, B0)


def _blockdiag(a):
    H_, ch = a.shape
    eye = jnp.eye(H_, dtype=jnp.float32)
    return (a[:, :, None] * eye[:, None, :]).reshape(H_ * ch, H_)


def _sel_pair(a_src, a_dst):
    bs, bd = _blockdiag(a_src), _blockdiag(a_dst)
    H_ = a_src.shape[0]
    pad = jnp.zeros((bs.shape[0], 16 - 2 * H_), jnp.float32)
    Ms = jnp.concatenate([bs, bd, pad], axis=1)
    Md = jnp.concatenate([bd, bs, pad], axis=1)
    return Ms, Md


def _pad_rows(a, rows):
    return jnp.pad(a, ((0, rows - a.shape[0]),) + ((0, 0),) * (a.ndim - 1))


def _chunked_edges(edge_index, totp, c, sub):
    loop = jnp.arange(N, dtype=jnp.int32)
    padi = jnp.full((totp - TOTE,), N, jnp.int32)
    s = jnp.concatenate([edge_index[0].astype(jnp.int32), loop, padi])
    d = jnp.concatenate([edge_index[1].astype(jnp.int32), loop, padi])
    sd = jnp.stack([s, d], 0).reshape(2, totp // c, sub, c // sub)
    return sd.transpose(1, 0, 2, 3)


def kernel(x, edge_index, batch, W1, a_src1, a_dst1, b1, W2, a_src2, a_dst2, b2):
    sd1 = _chunked_edges(edge_index, TOTP1, C1, SUB1)
    sd2 = _chunked_edges(edge_index, TOTP2, C2, SUB2)
    z16 = jnp.zeros((NPAD, 16), jnp.float32)
    z128 = jnp.zeros((NPAD, 128), jnp.float32)
    z32 = jnp.zeros((NPAD, 32), jnp.float32)

    Ms1, Md1 = _sel_pair(a_src1, a_dst1)
    h1, tS1, tD1 = _stage_a(x, W1, Ms1, Md1)
    den1, acc1 = _make_sc_edge(128, HEADS, C1, SUB1, CPW1)(
        sd1, _pad_rows(tS1, NPAD), _pad_rows(tD1, NPAD),
        _pad_rows(h1, NPAD), z16, z128)

    hsel = lax.broadcasted_iota(jnp.int32, (16, 128), 0)
    csel = lax.broadcasted_iota(jnp.int32, (16, 128), 1) // HID
    R = jnp.where(hsel == csel, 1.0, 0.0)
    Ms2, Md2 = _sel_pair(a_src2, a_dst2)
    h2, tS2, tD2 = _stage_b(den1, acc1, R, b1.reshape(1, 128), W2, Ms2, Md2)

    (acc2,) = _make_sc_edge(16, 1, C2, SUB2, CPW2)(sd2, tS2, tD2, h2, z16, z32)

    B0 = jnp.where(lax.broadcasted_iota(jnp.int32, (16, 16), 0) == 0, 1.0, 0.0)
    return _stage_c(acc2, b2.reshape(1, DOUT), batch.reshape(1, N), B0)


# speedup vs baseline: 1.1969x; 1.1484x over previous
import functools

import jax
import jax.numpy as jnp
from jax import lax
from jax.experimental import pallas as pl
from jax.experimental.pallas import tpu as pltpu
from jax.experimental.pallas import tpu_sc as plsc

N = 10000
E = 320000
DIN = 128
HID = 16
HEADS = 8
DOUT = 16
NG = 64
NEG_SLOPE = 0.2

NPAD = 10112
RPT = NPAD // 16
TOTE = E + N
NW = 32

C1, SUB1, CPW1 = 80, 1, 129
C2, SUB2, CPW2 = 512, 4, 21
TOTP1 = NW * CPW1 * C1
TOTP2 = NW * CPW2 * C2


def _stage_a_body(x_ref, w_ref, ms_ref, md_ref, h_ref, ts_ref, td_ref):
    h = jnp.dot(x_ref[...], w_ref[...], preferred_element_type=jnp.float32)
    h_ref[...] = h
    ts_ref[...] = jnp.dot(h, ms_ref[...], preferred_element_type=jnp.float32)
    td_ref[...] = jnp.dot(h, md_ref[...], preferred_element_type=jnp.float32)


def _stage_a(x, W, Ms, Md):
    n = x.shape[0]
    dout = W.shape[1]
    return pl.pallas_call(
        _stage_a_body,
        out_shape=(
            jax.ShapeDtypeStruct((n, dout), jnp.float32),
            jax.ShapeDtypeStruct((n, 16), jnp.float32),
            jax.ShapeDtypeStruct((n, 16), jnp.float32),
        ),
    )(x, W, Ms, Md)


def _make_sc_edge(D, H, C, SUB, CPW):
    CS = C // SUB
    MERGE = (D + 16) <= 128
    DW = D + 16 if MERGE else D
    mesh = plsc.VectorSubcoreMesh(core_axis_name="c", subcore_axis_name="s")

    def body(sd_h, tabS_h, tabD_h, htab_h, z16_h, zD_h, *refs):
        if MERGE:
            (acc_o, sd, dsc, bufS, bufD, hbuf, wbuf, acc_sh, *sems) = refs
        else:
            (den_o, acc_o, sd, dsc, bufS, bufD, hbuf, den_sh, acc_sh,
             *sems) = refs
        cid = lax.axis_index("c")
        sid = lax.axis_index("s")
        wid = cid * 16 + sid
        row0 = pl.multiple_of(sid * RPT, 8)
        pltpu.sync_copy(zD_h.at[pl.ds(row0, RPT)], acc_sh.at[pl.ds(row0, RPT)])
        if not MERGE:
            pltpu.sync_copy(z16_h.at[pl.ds(row0, RPT)],
                            den_sh.at[pl.ds(row0, RPT)])
        else:
            zv = jnp.zeros((16,), jnp.float32)

            def zi(i, c2):
                for sl in range(3):
                    wbuf[sl, i, pl.ds(D, 16)] = zv
                return c2

            lax.fori_loop(0, C, zi, 0)
        plsc.subcore_barrier()

        lane = lax.iota(jnp.int32, 16)
        lane_ok = lane < H
        ck0 = wid * CPW
        isems = sems[0:3]
        dsems = sems[3:6]
        gsems = sems[6:9]
        ssems = sems[9:12]

        def idx_start(k, slot):
            pltpu.async_copy(sd_h.at[ck0 + k], sd.at[slot], isems[slot])

        def idx_wait(slot):
            pltpu.make_async_copy(sd_h.at[0], sd.at[slot], isems[slot]).wait()

        def dsc_start(k, slot):
            pltpu.async_copy(sd_h.at[ck0 + k, 1], dsc.at[slot], dsems[slot])

        def dsc_wait(slot):
            pltpu.make_async_copy(sd_h.at[0, 1], dsc.at[slot],
                                  dsems[slot]).wait()

        def gathers_start(slot):
            for s in range(SUB):
                r = pl.ds(s * CS, CS)
                pltpu.async_copy(tabS_h.at[sd.at[slot, 0, s]],
                                 bufS.at[slot, r], gsems[slot])
                pltpu.async_copy(tabD_h.at[sd.at[slot, 1, s]],
                                 bufD.at[slot, r], gsems[slot])
                pltpu.async_copy(htab_h.at[sd.at[slot, 0, s]],
                                 hbuf.at[slot, r], gsems[slot])

        def gathers_wait(slot):
            for s in range(SUB):
                r = pl.ds(s * CS, CS)
                pltpu.make_async_copy(tabS_h.at[sd.at[slot, 0, s]],
                                      bufS.at[slot, r], gsems[slot]).wait()
                pltpu.make_async_copy(tabD_h.at[sd.at[slot, 1, s]],
                                      bufD.at[slot, r], gsems[slot]).wait()
                pltpu.make_async_copy(htab_h.at[sd.at[slot, 0, s]],
                                      hbuf.at[slot, r], gsems[slot]).wait()

        if MERGE:
            def scatters_start(slot):
                for s in range(SUB):
                    pltpu.async_copy(wbuf.at[slot, pl.ds(s * CS, CS)],
                                     acc_sh.at[dsc.at[slot, s]],
                                     ssems[slot], add=True)

            def scatters_wait(slot):
                for s in range(SUB):
                    pltpu.make_async_copy(wbuf.at[slot, pl.ds(s * CS, CS)],
                                          acc_sh.at[dsc.at[slot, s]],
                                          ssems[slot]).wait()

            def compute(slot):
                def cl(g, c2):
                    i0 = g * 16
                    sv = bufS[slot, pl.ds(i0, 16)]
                    dv = bufD[slot, pl.ds(i0, 16)]
                    e = sv + dv
                    e = jnp.where(e > 0.0, e, NEG_SLOPE * e)
                    ex = jnp.exp(e)
                    plsc.store_scatter(
                        wbuf,
                        [jnp.full((16,), slot, jnp.int32), i0 + lane,
                         jnp.full((16,), D, jnp.int32)], ex)
                    for j in range(16):
                        hv = hbuf[slot, i0 + j, :]
                        wbuf[slot, i0 + j, pl.ds(0, 16)] = hv * jnp.full(
                            (16,), ex[j], jnp.float32)
                    return c2

                lax.fori_loop(0, C // 16, cl, 0)
        else:
            def scatters_start(slot):
                for s in range(SUB):
                    r = pl.ds(s * CS, CS)
                    pltpu.async_copy(bufS.at[slot, r],
                                     den_sh.at[dsc.at[slot, s]],
                                     ssems[slot], add=True)
                    pltpu.async_copy(hbuf.at[slot, r],
                                     acc_sh.at[dsc.at[slot, s]],
                                     ssems[slot], add=True)

            def scatters_wait(slot):
                for s in range(SUB):
                    r = pl.ds(s * CS, CS)
                    pltpu.make_async_copy(bufS.at[slot, r],
                                          den_sh.at[dsc.at[slot, s]],
                                          ssems[slot]).wait()
                    pltpu.make_async_copy(hbuf.at[slot, r],
                                          acc_sh.at[dsc.at[slot, s]],
                                          ssems[slot]).wait()

            def compute(slot):
                def cl(i, c2):
                    e = bufS[slot, i, :] + bufD[slot, i, :]
                    e = jnp.where(e > 0.0, e, NEG_SLOPE * e)
                    ex = jnp.where(lane_ok, jnp.exp(e), 0.0)
                    bufS[slot, i, :] = ex
                    for j in range(D // 16):
                        hv = hbuf[slot, i, pl.ds(j * 16, 16)]
                        hbuf[slot, i, pl.ds(j * 16, 16)] = hv * jnp.full(
                            (16,), ex[j], jnp.float32)
                    return c2

                lax.fori_loop(0, C, cl, 0, unroll=2)

        def handle(k, slot):
            sn = (slot + 1) % 3
            sp = (slot + 2) % 3

            @pl.when(k + 1 < CPW)
            def _():
                idx_wait(sn)
                gathers_start(sn)

            gathers_wait(slot)

            @pl.when(k + 2 < CPW)
            def _():
                idx_start(k + 2, sp)

            compute(slot)

            @pl.when(k >= 1)
            def _():
                scatters_wait(sp)

            @pl.when(k + 2 < CPW)
            def _():
                dsc_start(k + 2, sp)

            dsc_wait(slot)
            scatters_start(slot)

        idx_start(0, 0)
        idx_start(1, 1)
        dsc_start(0, 0)
        dsc_start(1, 1)
        idx_wait(0)
        gathers_start(0)

        def triple(g, carry):
            handle(3 * g, 0)
            handle(3 * g + 1, 1)
            handle(3 * g + 2, 2)
            return carry

        lax.fori_loop(0, CPW // 3, triple, 0)
        scatters_wait((CPW - 1) % 3)

        plsc.subcore_barrier()
        pltpu.sync_copy(acc_sh.at[pl.ds(row0, RPT)],
                        acc_o.at[cid, pl.ds(row0, RPT)])
        if not MERGE:
            pltpu.sync_copy(den_sh.at[pl.ds(row0, RPT)],
                            den_o.at[cid, pl.ds(row0, RPT)])

    out_type = [jax.ShapeDtypeStruct((2, NPAD, DW), jnp.float32)]
    bufshape = (3, C) if MERGE else (3, C, 16)
    scratch = [
        pltpu.VMEM((3, 2, SUB, CS), jnp.int32),
        pltpu.VMEM((3, SUB, CS), jnp.int32),
        pltpu.VMEM(bufshape, jnp.float32),
        pltpu.VMEM(bufshape, jnp.float32),
        pltpu.VMEM((3, C, D), jnp.float32),
    ]
    if MERGE:
        scratch += [
            pltpu.VMEM((3, C, DW), jnp.float32),
            pltpu.VMEM_SHARED((NPAD, DW), jnp.float32),
        ]
    else:
        out_type = [jax.ShapeDtypeStruct((2, NPAD, 16), jnp.float32)] + out_type
        scratch += [
            pltpu.VMEM_SHARED((NPAD, 16), jnp.float32),
            pltpu.VMEM_SHARED((NPAD, D), jnp.float32),
        ]
    scratch += [pltpu.SemaphoreType.DMA] * 12

    return pl.kernel(
        body,
        out_type=out_type,
        mesh=mesh,
        scratch_types=scratch,
        compiler_params=pltpu.CompilerParams(use_tc_tiling_on_sc=False, needs_layout_passes=False),
    )


def _stage_b_body(den_r, acc_r, R_r, b1_r, W2_r, Ms_r, Md_r, h2_o, ts_o, td_o):
    densum = den_r[0] + den_r[1]
    accsum = acc_r[0] + acc_r[1]
    dexp = jnp.dot(densum, R_r[...], preferred_element_type=jnp.float32)
    g1 = accsum / (dexp + 1e-16) + b1_r[...]
    g1 = jnp.where(g1 > 0.0, g1, jnp.exp(jnp.minimum(g1, 0.0)) - 1.0)
    h2 = jnp.dot(g1, W2_r[...], preferred_element_type=jnp.float32)
    h2_o[...] = h2
    ts_o[...] = jnp.dot(h2, Ms_r[...], preferred_element_type=jnp.float32)
    td_o[...] = jnp.dot(h2, Md_r[...], preferred_element_type=jnp.float32)


def _stage_b(den1, acc1, R, b1r, W2, Ms2, Md2):
    return pl.pallas_call(
        _stage_b_body,
        out_shape=(
            jax.ShapeDtypeStruct((NPAD, 16), jnp.float32),
            jax.ShapeDtypeStruct((NPAD, 16), jnp.float32),
            jax.ShapeDtypeStruct((NPAD, 16), jnp.float32),
        ),
    )(den1, acc1, R, b1r, W2, Ms2, Md2)


def _stage_c_body(acc_r, b2_r, batch_r, B0_r, out_o):
    accsum = acc_r[0] + acc_r[1]
    densum = accsum[:, 16:32]
    dl = jnp.dot(densum, B0_r[...], preferred_element_type=jnp.float32)
    g2full = accsum[:, 0:16] / (dl + 1e-16)
    g2 = g2full[0:N, :] + b2_r[...]
    gid = lax.broadcasted_iota(jnp.int32, (NG, N), 0)
    bm = jnp.where(gid == batch_r[...], 1.0, 0.0)
    pooled = jnp.dot(bm, g2, preferred_element_type=jnp.float32)
    cnt = jnp.sum(bm, axis=1, keepdims=True)
    pooled = pooled / jnp.maximum(cnt, 1.0)
    m = jnp.max(pooled, axis=1, keepdims=True)
    p = pooled - m
    lse = jnp.log(jnp.sum(jnp.exp(p), axis=1, keepdims=True))
    out_o[...] = p - lse


def _stage_c(acc2, b2r, batch2, B0):
    return pl.pallas_call(
        _stage_c_body,
        out_shape=jax.ShapeDtypeStruct((NG, DOUT), jnp.float32),
    )(acc2, b2r, batch2, B0)


def _blockdiag(a):
    H_, ch = a.shape
    eye = jnp.eye(H_, dtype=jnp.float32)
    return (a[:, :, None] * eye[:, None, :]).reshape(H_ * ch, H_)


def _sel_pair(a_src, a_dst):
    bs, bd = _blockdiag(a_src), _blockdiag(a_dst)
    H_ = a_src.shape[0]
    pad = jnp.zeros((bs.shape[0], 16 - 2 * H_), jnp.float32)
    Ms = jnp.concatenate([bs, bd, pad], axis=1)
    Md = jnp.concatenate([bd, bs, pad], axis=1)
    return Ms, Md


def _pad_rows(a, rows):
    return jnp.pad(a, ((0, rows - a.shape[0]),) + ((0, 0),) * (a.ndim - 1))


def _chunked_edges(edge_index, totp, c, sub):
    loop = jnp.arange(N, dtype=jnp.int32)
    padi = jnp.full((totp - TOTE,), N, jnp.int32)
    s = jnp.concatenate([edge_index[0].astype(jnp.int32), loop, padi])
    d = jnp.concatenate([edge_index[1].astype(jnp.int32), loop, padi])
    sd = jnp.stack([s, d], 0).reshape(2, totp // c, sub, c // sub)
    return sd.transpose(1, 0, 2, 3)


def kernel(x, edge_index, batch, W1, a_src1, a_dst1, b1, W2, a_src2, a_dst2, b2):
    sd1 = _chunked_edges(edge_index, TOTP1, C1, SUB1)
    sd2 = _chunked_edges(edge_index, TOTP2, C2, SUB2)
    z16 = jnp.zeros((NPAD, 16), jnp.float32)
    z128 = jnp.zeros((NPAD, 128), jnp.float32)
    z32 = jnp.zeros((NPAD, 32), jnp.float32)

    Ms1, Md1 = _sel_pair(a_src1, a_dst1)
    h1, tS1, tD1 = _stage_a(x, W1, Ms1, Md1)
    den1, acc1 = _make_sc_edge(128, HEADS, C1, SUB1, CPW1)(
        sd1, _pad_rows(tS1, NPAD), _pad_rows(tD1, NPAD),
        _pad_rows(h1, NPAD), z16, z128)

    hsel = lax.broadcasted_iota(jnp.int32, (16, 128), 0)
    csel = lax.broadcasted_iota(jnp.int32, (16, 128), 1) // HID
    R = jnp.where(hsel == csel, 1.0, 0.0)
    Ms2, Md2 = _sel_pair(a_src2, a_dst2)
    h2, tS2, tD2 = _stage_b(den1, acc1, R, b1.reshape(1, 128), W2, Ms2, Md2)

    (acc2,) = _make_sc_edge(16, 1, C2, SUB2, CPW2)(
        sd2, tS2[:, 0], tD2[:, 0], h2, z16, z32)

    B0 = jnp.where(lax.broadcasted_iota(jnp.int32, (16, 16), 0) == 0, 1.0, 0.0)
    return _stage_c(acc2, b2.reshape(1, DOUT), batch.reshape(1, N), B0)
